# baseline (speedup 1.0000x reference)
import functools

import jax
import jax.numpy as jnp
from jax.experimental import pallas as pl
from jax.experimental.pallas import tpu as pltpu

_HIDDEN = 512
_SUBSEQ = 200
_K = 5
_SORT_W = 1024

_MLP_BLK = 512
_RANK_BLK = 256
_MERGE_BLK = 512
_KNN_BLK = 256
_SORT_BLK = 256


def _mlp_kernel(z_ref, w1_ref, b1_ref, w2_ref, b2_ref, out_ref):
    h = jnp.dot(z_ref[...], w1_ref[...], preferred_element_type=jnp.float32)
    h = jnp.maximum(h + b1_ref[...], 0.0)
    out_ref[...] = jnp.dot(h, w2_ref[...], preferred_element_type=jnp.float32) + b2_ref[...]


def _zsim_kernel(n, blk, zp_ref, zr_ref, out_ref):
    i = pl.program_id(0)
    s = jnp.sum(zp_ref[...] * zr_ref[...], axis=1, keepdims=True)
    rows = i * blk + jax.lax.broadcasted_iota(jnp.int32, (blk, 1), 0)
    out_ref[...] = jnp.where(rows == n - 1, -100000000.0, s)


def _rank_kernel(n, blk, comp_count, scol_ref, srow_ref, out_ref):
    i = pl.program_id(0)
    sc = scol_ref[...]
    sr = srow_ref[...]
    jidx = jax.lax.broadcasted_iota(jnp.int32, (1, n), 1)
    iidx = i * blk + jax.lax.broadcasted_iota(jnp.int32, (blk, 1), 0)
    lt = (sr < sc).astype(jnp.float32)
    eqlt = ((sr == sc) & (jidx < iidx)).astype(jnp.float32)
    cnt = jnp.sum(lt + eqlt, axis=1, keepdims=True)
    out_ref[...] = (cnt < float(comp_count)).astype(jnp.float32)


def _merge_kernel(z_ref, zr_ref, y_ref, yr_ref, comp_ref, zn_ref, ym_ref):
    comp = comp_ref[...] > 0.5
    z = z_ref[...]
    zm = jnp.where(comp, z, (z + zr_ref[...]) / 2.0)
    nrm = jnp.sqrt(jnp.sum(zm * zm, axis=1, keepdims=True))
    zn_ref[...] = zm / jnp.maximum(nrm, 1e-12)
    y = y_ref[...]
    ym_ref[...] = jnp.where(comp, y, jnp.minimum(y, yr_ref[...]))


def _knn_kernel(n, blk, zi_ref, zall_ref, vals_ref, idxs_ref):
    a = jax.lax.dot_general(
        zi_ref[...], zall_ref[...], (((1,), (1,)), ((), ())),
        preferred_element_type=jnp.float32)
    lane = jax.lax.broadcasted_iota(jnp.int32, (blk, n), 1)
    vals, idxs = [], []
    for _ in range(_K + 1):
        m = jnp.max(a, axis=1, keepdims=True)
        am = jnp.min(jnp.where(a == m, lane, n), axis=1, keepdims=True)
        vals.append(m)
        idxs.append(am)
        a = jnp.where(lane == am, -jnp.inf, a)
    vals_ref[...] = jnp.concatenate(vals, axis=1)
    idxs_ref[...] = jnp.concatenate(idxs, axis=1)


def _partner(x, d, low):
    up = pltpu.roll(x, _SORT_W - d, 1)
    dn = pltpu.roll(x, d, 1)
    return jnp.where(low, up, dn)


def _sort_kernel(l, blk, x_ref, xr_ref, m_ref, mr_ref, comp_ref, xm_ref, mm_ref):
    lane = jax.lax.broadcasted_iota(jnp.int32, (blk, _SORT_W), 1)
    comp = comp_ref[...] > 0.5
    pad = _SORT_W - 2 * l
    key = jnp.concatenate(
        [m_ref[...], mr_ref[...], jnp.full((blk, pad), -1.0, jnp.float32)], axis=1)
    zero_zone = comp & (lane >= 2 * l - _SUBSEQ) & (lane < 2 * l)
    key = jnp.where(zero_zone, 0.0, key)
    pay = jnp.concatenate(
        [x_ref[...], xr_ref[...], jnp.zeros((blk, pad), jnp.float32)], axis=1)
    idx = lane

    kk = 2
    while kk <= _SORT_W:
        desc = (lane & kk) == 0
        d = kk // 2
        while d >= 1:
            low = (lane & d) == 0
            pk = _partner(key, d, low)
            pi = _partner(idx, d, low)
            pp = _partner(pay, d, low)
            gt = (key > pk) | ((key == pk) & (idx < pi))
            take_self = gt == (desc == low)
            key = jnp.where(take_self, key, pk)
            idx = jnp.where(take_self, idx, pi)
            pay = jnp.where(take_self, pay, pp)
            d //= 2
        kk *= 2

    xm_ref[...] = pay[:, :2 * l]
    mm_ref[...] = key[:, :2 * l]


def kernel(z, x, y, x_mask, temporal_edge_index, temporal_edge_attr,
           sliding_wdw, W1, b1, W2, b2):
    n, dd = z.shape
    l = x.shape[1]
    merge_num = n // 2
    f32 = jnp.float32

    z_proj = pl.pallas_call(
        _mlp_kernel,
        grid=(n // _MLP_BLK,),
        in_specs=[
            pl.BlockSpec((_MLP_BLK, dd), lambda i: (i, 0)),
            pl.BlockSpec((dd, _HIDDEN), lambda i: (0, 0)),
            pl.BlockSpec((1, _HIDDEN), lambda i: (0, 0)),
            pl.BlockSpec((_HIDDEN, _HIDDEN), lambda i: (0, 0)),
            pl.BlockSpec((1, _HIDDEN), lambda i: (0, 0)),
        ],
        out_specs=pl.BlockSpec((_MLP_BLK, _HIDDEN), lambda i: (i, 0)),
        out_shape=jax.ShapeDtypeStruct((n, _HIDDEN), f32),
    )(z, W1, b1.reshape(1, -1), W2, b2.reshape(1, -1))

    zp_roll = jnp.roll(z_proj, -1, axis=0)
    z_sim = pl.pallas_call(
        functools.partial(_zsim_kernel, n, _MLP_BLK),
        grid=(n // _MLP_BLK,),
        in_specs=[
            pl.BlockSpec((_MLP_BLK, _HIDDEN), lambda i: (i, 0)),
            pl.BlockSpec((_MLP_BLK, _HIDDEN), lambda i: (i, 0)),
        ],
        out_specs=pl.BlockSpec((_MLP_BLK, 1), lambda i: (i, 0)),
        out_shape=jax.ShapeDtypeStruct((n, 1), f32),
    )(z_proj, zp_roll)

    comp = pl.pallas_call(
        functools.partial(_rank_kernel, n, _RANK_BLK, n - merge_num),
        grid=(n // _RANK_BLK,),
        in_specs=[
            pl.BlockSpec((_RANK_BLK, 1), lambda i: (i, 0)),
            pl.BlockSpec((1, n), lambda i: (0, 0)),
        ],
        out_specs=pl.BlockSpec((_RANK_BLK, 1), lambda i: (i, 0)),
        out_shape=jax.ShapeDtypeStruct((n, 1), f32),
    )(z_sim, z_sim.reshape(1, n))

    z_roll = jnp.roll(z, -1, axis=0)
    y2 = y.reshape(n, 1)
    y2_roll = jnp.roll(y2, -1, axis=0)
    z_n, y_merge2 = pl.pallas_call(
        _merge_kernel,
        grid=(n // _MERGE_BLK,),
        in_specs=[
            pl.BlockSpec((_MERGE_BLK, dd), lambda i: (i, 0)),
            pl.BlockSpec((_MERGE_BLK, dd), lambda i: (i, 0)),
            pl.BlockSpec((_MERGE_BLK, 1), lambda i: (i, 0)),
            pl.BlockSpec((_MERGE_BLK, 1), lambda i: (i, 0)),
            pl.BlockSpec((_MERGE_BLK, 1), lambda i: (i, 0)),
        ],
        out_specs=[
            pl.BlockSpec((_MERGE_BLK, dd), lambda i: (i, 0)),
            pl.BlockSpec((_MERGE_BLK, 1), lambda i: (i, 0)),
        ],
        out_shape=[
            jax.ShapeDtypeStruct((n, dd), f32),
            jax.ShapeDtypeStruct((n, 1), f32),
        ],
    )(z, z_roll, y2, y2_roll, comp)

    vals, idxs = pl.pallas_call(
        functools.partial(_knn_kernel, n, _KNN_BLK),
        grid=(n // _KNN_BLK,),
        in_specs=[
            pl.BlockSpec((_KNN_BLK, dd), lambda i: (i, 0)),
            pl.BlockSpec((n, dd), lambda i: (0, 0)),
        ],
        out_specs=[
            pl.BlockSpec((_KNN_BLK, _K + 1), lambda i: (i, 0)),
            pl.BlockSpec((_KNN_BLK, _K + 1), lambda i: (i, 0)),
        ],
        out_shape=[
            jax.ShapeDtypeStruct((n, _K + 1), f32),
            jax.ShapeDtypeStruct((n, _K + 1), jnp.int32),
        ],
    )(z_n, z_n)

    x_roll = jnp.roll(x, -1, axis=0)
    m_roll = jnp.roll(x_mask, -1, axis=0)
    x_merge, x_mask_merge = pl.pallas_call(
        functools.partial(_sort_kernel, l, _SORT_BLK),
        grid=(n // _SORT_BLK,),
        in_specs=[
            pl.BlockSpec((_SORT_BLK, l), lambda i: (i, 0)),
            pl.BlockSpec((_SORT_BLK, l), lambda i: (i, 0)),
            pl.BlockSpec((_SORT_BLK, l), lambda i: (i, 0)),
            pl.BlockSpec((_SORT_BLK, l), lambda i: (i, 0)),
            pl.BlockSpec((_SORT_BLK, 1), lambda i: (i, 0)),
        ],
        out_specs=[
            pl.BlockSpec((_SORT_BLK, 2 * l), lambda i: (i, 0)),
            pl.BlockSpec((_SORT_BLK, 2 * l), lambda i: (i, 0)),
        ],
        out_shape=[
            jax.ShapeDtypeStruct((n, 2 * l), f32),
            jax.ShapeDtypeStruct((n, 2 * l), f32),
        ],
    )(x, x_roll, x_mask, m_roll, comp)

    dist = vals[:, 1:]
    idx = idxs[:, 1:]
    idx_source = jnp.repeat(jnp.arange(n, dtype=jnp.int32), _K)
    edge_index = jnp.stack([idx_source, idx.reshape(-1)], axis=0)
    attr = dist.reshape(-1, 1)
    return (x_merge, edge_index, attr, y_merge2.reshape(n),
            temporal_edge_index, temporal_edge_attr, x_mask_merge)


# speedup vs baseline: 2.0015x; 2.0015x over previous
import functools

import jax
import jax.numpy as jnp
from jax import lax
from jax.experimental import pallas as pl
from jax.experimental.pallas import tpu as pltpu
from jax.experimental.pallas import tpu_sc as plsc

_HIDDEN = 512
_SUBSEQ = 200
_K = 5

_MLP_BLK = 512
_RANK_BLK = 256
_MERGE_BLK = 512
_KNN_BLK = 256

_N = 4096
_L = 400
_W = 2 * _L
_NW = 32
_RPW = _N // _NW
_BATCH = 16
_NB = _RPW // _BATCH
_NCH = _W // 16
_KMAX = 0x3F7FFFFF


def _mlp_kernel(z_ref, w1_ref, b1_ref, w2_ref, b2_ref, out_ref):
    h = jnp.dot(z_ref[...], w1_ref[...], preferred_element_type=jnp.float32)
    h = jnp.maximum(h + b1_ref[...], 0.0)
    out_ref[...] = jnp.dot(h, w2_ref[...], preferred_element_type=jnp.float32) + b2_ref[...]


def _zsim_kernel(n, blk, zp_ref, zr_ref, out_ref):
    i = pl.program_id(0)
    s = jnp.sum(zp_ref[...] * zr_ref[...], axis=1, keepdims=True)
    rows = i * blk + jax.lax.broadcasted_iota(jnp.int32, (blk, 1), 0)
    out_ref[...] = jnp.where(rows == n - 1, -100000000.0, s)


def _rank_kernel(n, blk, comp_count, scol_ref, srow_ref, out_ref):
    i = pl.program_id(0)
    sc = scol_ref[...]
    sr = srow_ref[...]
    jidx = jax.lax.broadcasted_iota(jnp.int32, (1, n), 1)
    iidx = i * blk + jax.lax.broadcasted_iota(jnp.int32, (blk, 1), 0)
    lt = (sr < sc).astype(jnp.float32)
    eqlt = ((sr == sc) & (jidx < iidx)).astype(jnp.float32)
    cnt = jnp.sum(lt + eqlt, axis=1, keepdims=True)
    out_ref[...] = (cnt < float(comp_count)).astype(jnp.float32)


def _merge_kernel(z_ref, zr_ref, y_ref, yr_ref, comp_ref, zn_ref, ym_ref):
    comp = comp_ref[...] > 0.5
    z = z_ref[...]
    zm = jnp.where(comp, z, (z + zr_ref[...]) / 2.0)
    nrm = jnp.sqrt(jnp.sum(zm * zm, axis=1, keepdims=True))
    zn_ref[...] = zm / jnp.maximum(nrm, 1e-12)
    y = y_ref[...]
    ym_ref[...] = jnp.where(comp, y, jnp.minimum(y, yr_ref[...]))


def _knn_kernel(n, blk, zi_ref, zall_ref, vals_ref, idxs_ref):
    a = jax.lax.dot_general(
        zi_ref[...], zall_ref[...], (((1,), (1,)), ((), ())),
        preferred_element_type=jnp.float32)
    lane = jax.lax.broadcasted_iota(jnp.int32, (blk, n), 1)
    vals, idxs = [], []
    for _ in range(_K + 1):
        m = jnp.max(a, axis=1, keepdims=True)
        am = jnp.min(jnp.where(a == m, lane, n), axis=1, keepdims=True)
        vals.append(m)
        idxs.append(am)
        a = jnp.where(lane == am, -jnp.inf, a)
    vals_ref[...] = jnp.concatenate(vals, axis=1)
    idxs_ref[...] = jnp.concatenate(idxs, axis=1)


_SC_MESH = plsc.VectorSubcoreMesh(core_axis_name="c", subcore_axis_name="s")


@functools.partial(
    pl.kernel, mesh=_SC_MESH,
    out_type=[
        jax.ShapeDtypeStruct((_N * _W,), jnp.float32),
        jax.ShapeDtypeStruct((_N * _W,), jnp.float32),
    ],
    scratch_types=[
        pltpu.VMEM(((_BATCH + 1) * _L,), jnp.float32),
        pltpu.VMEM(((_BATCH + 1) * _L,), jnp.float32),
        pltpu.VMEM((_BATCH * _W,), jnp.float32),
        pltpu.VMEM((_BATCH * _W,), jnp.float32),
        pltpu.VMEM((_W,), jnp.int32),
        pltpu.VMEM((_W,), jnp.float32),
        pltpu.VMEM((_W,), jnp.int32),
        pltpu.VMEM((_W,), jnp.float32),
        pltpu.VMEM((256,), jnp.int32),
        pltpu.VMEM((256,), jnp.int32),
        pltpu.VMEM((_RPW,), jnp.float32),
    ],
    compiler_params=pltpu.CompilerParams(needs_layout_passes=False),
)
def _sc_sort(x_hbm, m_hbm, comp_hbm, outk_hbm, outp_hbm,
             m_buf, x_buf, outk_buf, outp_buf, a_k, a_p, b_k, b_p, cnt, rr, comp_v):
    wid = lax.axis_index("s") * 2 + lax.axis_index("c")
    iota = jax.lax.iota(jnp.int32, 16)
    ones = jnp.ones(16, jnp.int32)
    zeros16 = jnp.zeros(16, jnp.int32)

    pltpu.sync_copy(comp_hbm.at[pl.ds(wid * _RPW, _RPW)], comp_v)

    def zero_cnt(i, c):
        cnt[pl.ds(i * 16, 16)] = zeros16
        return c

    def prefix(i, carry):
        v = cnt[pl.ds(i * 16, 16)]
        inc = plsc.cumsum(v)
        rr[pl.ds(i * 16, 16)] = inc - v + carry
        return carry + jnp.sum(v)

    def batch_body(rb, carry0):
        r0 = wid * _RPW + rb * _BATCH
        re = lax.rem(r0 + _BATCH, _N)
        pltpu.sync_copy(m_hbm.at[pl.ds(r0 * _L, _BATCH * _L)], m_buf.at[pl.ds(0, _BATCH * _L)])
        pltpu.sync_copy(m_hbm.at[pl.ds(re * _L, _L)], m_buf.at[pl.ds(_BATCH * _L, _L)])
        pltpu.sync_copy(x_hbm.at[pl.ds(r0 * _L, _BATCH * _L)], x_buf.at[pl.ds(0, _BATCH * _L)])
        pltpu.sync_copy(x_hbm.at[pl.ds(re * _L, _L)], x_buf.at[pl.ds(_BATCH * _L, _L)])

        def row_body(rl, rcarry):
            mo = rl * _L
            oo = rl * _W
            cv = plsc.load_gather(comp_v, [zeros16 + (rb * _BATCH + rl)])
            is_comp = cv > 0.5

            lax.fori_loop(0, 16, zero_cnt, 0)

            def count0(c, cy):
                raw = m_buf[pl.ds(mo + c * 16, 16)]
                gl = c * 16 + iota
                zc = is_comp & (gl >= _W - _SUBSEQ)
                val = jnp.where(zc, 0.0, raw)
                tk = _KMAX - plsc.bitcast(val, jnp.int32)
                a_k[pl.ds(c * 16, 16)] = tk
                plsc.addupdate_scatter(cnt, [tk & 255], ones)
                return cy
            lax.fori_loop(0, _NCH, count0, 0)
            lax.fori_loop(0, 16, prefix, jnp.int32(0))

            def scat0(c, cy):
                tk = a_k[pl.ds(c * 16, 16)]
                pv = x_buf[pl.ds(mo + c * 16, 16)]
                dig = tk & 255
                c1, lastm = plsc.scan_count(dig)
                pos = plsc.load_gather(rr, [dig]) + c1 - 1
                plsc.store_scatter(b_k, [pos], tk)
                plsc.store_scatter(b_p, [pos], pv)
                plsc.addupdate_scatter(rr, [dig], c1, mask=lastm)
                return cy
            lax.fori_loop(0, _NCH, scat0, 0)

            for pno, (sk, sp, dk, dp) in enumerate(
                    [(b_k, b_p, a_k, a_p), (a_k, a_p, b_k, b_p)], start=1):
                shift = 8 * pno
                lax.fori_loop(0, 16, zero_cnt, 0)

                def countp(c, cy, sk=sk, shift=shift):
                    tk = sk[pl.ds(c * 16, 16)]
                    plsc.addupdate_scatter(
                        cnt, [lax.shift_right_logical(tk, shift) & 255], ones)
                    return cy
                lax.fori_loop(0, _NCH, countp, 0)
                lax.fori_loop(0, 16, prefix, jnp.int32(0))

                def scat(c, cy, sk=sk, sp=sp, dk=dk, dp=dp, shift=shift):
                    tk = sk[pl.ds(c * 16, 16)]
                    pv = sp[pl.ds(c * 16, 16)]
                    dig = lax.shift_right_logical(tk, shift) & 255
                    c1, lastm = plsc.scan_count(dig)
                    pos = plsc.load_gather(rr, [dig]) + c1 - 1
                    plsc.store_scatter(dk, [pos], tk)
                    plsc.store_scatter(dp, [pos], pv)
                    plsc.addupdate_scatter(rr, [dig], c1, mask=lastm)
                    return cy
                lax.fori_loop(0, _NCH, scat, 0)

            lax.fori_loop(0, 16, zero_cnt, 0)

            def count3(c, cy):
                tk = b_k[pl.ds(c * 16, 16)]
                plsc.addupdate_scatter(
                    cnt, [lax.shift_right_logical(tk, 24) & 255], ones)
                return cy
            lax.fori_loop(0, _NCH, count3, 0)
            lax.fori_loop(0, 16, prefix, jnp.int32(0))

            def scat3(c, cy):
                tk = b_k[pl.ds(c * 16, 16)]
                pv = b_p[pl.ds(c * 16, 16)]
                dig = lax.shift_right_logical(tk, 24) & 255
                c1, lastm = plsc.scan_count(dig)
                pos = oo + plsc.load_gather(rr, [dig]) + c1 - 1
                plsc.store_scatter(outk_buf, [pos], plsc.bitcast(_KMAX - tk, jnp.float32))
                plsc.store_scatter(outp_buf, [pos], pv)
                plsc.addupdate_scatter(rr, [dig], c1, mask=lastm)
                return cy
            lax.fori_loop(0, _NCH, scat3, 0)
            return rcarry

        lax.fori_loop(0, _BATCH, row_body, 0)
        pltpu.sync_copy(outk_buf, outk_hbm.at[pl.ds(r0 * _W, _BATCH * _W)])
        pltpu.sync_copy(outp_buf, outp_hbm.at[pl.ds(r0 * _W, _BATCH * _W)])
        return carry0

    lax.fori_loop(0, _NB, batch_body, 0)


def kernel(z, x, y, x_mask, temporal_edge_index, temporal_edge_attr,
           sliding_wdw, W1, b1, W2, b2):
    n, dd = z.shape
    l = x.shape[1]
    merge_num = n // 2
    f32 = jnp.float32

    z_proj = pl.pallas_call(
        _mlp_kernel,
        grid=(n // _MLP_BLK,),
        in_specs=[
            pl.BlockSpec((_MLP_BLK, dd), lambda i: (i, 0)),
            pl.BlockSpec((dd, _HIDDEN), lambda i: (0, 0)),
            pl.BlockSpec((1, _HIDDEN), lambda i: (0, 0)),
            pl.BlockSpec((_HIDDEN, _HIDDEN), lambda i: (0, 0)),
            pl.BlockSpec((1, _HIDDEN), lambda i: (0, 0)),
        ],
        out_specs=pl.BlockSpec((_MLP_BLK, _HIDDEN), lambda i: (i, 0)),
        out_shape=jax.ShapeDtypeStruct((n, _HIDDEN), f32),
    )(z, W1, b1.reshape(1, -1), W2, b2.reshape(1, -1))

    zp_roll = jnp.roll(z_proj, -1, axis=0)
    z_sim = pl.pallas_call(
        functools.partial(_zsim_kernel, n, _MLP_BLK),
        grid=(n // _MLP_BLK,),
        in_specs=[
            pl.BlockSpec((_MLP_BLK, _HIDDEN), lambda i: (i, 0)),
            pl.BlockSpec((_MLP_BLK, _HIDDEN), lambda i: (i, 0)),
        ],
        out_specs=pl.BlockSpec((_MLP_BLK, 1), lambda i: (i, 0)),
        out_shape=jax.ShapeDtypeStruct((n, 1), f32),
    )(z_proj, zp_roll)

    comp = pl.pallas_call(
        functools.partial(_rank_kernel, n, _RANK_BLK, n - merge_num),
        grid=(n // _RANK_BLK,),
        in_specs=[
            pl.BlockSpec((_RANK_BLK, 1), lambda i: (i, 0)),
            pl.BlockSpec((1, n), lambda i: (0, 0)),
        ],
        out_specs=pl.BlockSpec((_RANK_BLK, 1), lambda i: (i, 0)),
        out_shape=jax.ShapeDtypeStruct((n, 1), f32),
    )(z_sim, z_sim.reshape(1, n))

    z_roll = jnp.roll(z, -1, axis=0)
    y2 = y.reshape(n, 1)
    y2_roll = jnp.roll(y2, -1, axis=0)
    z_n, y_merge2 = pl.pallas_call(
        _merge_kernel,
        grid=(n // _MERGE_BLK,),
        in_specs=[
            pl.BlockSpec((_MERGE_BLK, dd), lambda i: (i, 0)),
            pl.BlockSpec((_MERGE_BLK, dd), lambda i: (i, 0)),
            pl.BlockSpec((_MERGE_BLK, 1), lambda i: (i, 0)),
            pl.BlockSpec((_MERGE_BLK, 1), lambda i: (i, 0)),
            pl.BlockSpec((_MERGE_BLK, 1), lambda i: (i, 0)),
        ],
        out_specs=[
            pl.BlockSpec((_MERGE_BLK, dd), lambda i: (i, 0)),
            pl.BlockSpec((_MERGE_BLK, 1), lambda i: (i, 0)),
        ],
        out_shape=[
            jax.ShapeDtypeStruct((n, dd), f32),
            jax.ShapeDtypeStruct((n, 1), f32),
        ],
    )(z, z_roll, y2, y2_roll, comp)

    vals, idxs = pl.pallas_call(
        functools.partial(_knn_kernel, n, _KNN_BLK),
        grid=(n // _KNN_BLK,),
        in_specs=[
            pl.BlockSpec((_KNN_BLK, dd), lambda i: (i, 0)),
            pl.BlockSpec((n, dd), lambda i: (0, 0)),
        ],
        out_specs=[
            pl.BlockSpec((_KNN_BLK, _K + 1), lambda i: (i, 0)),
            pl.BlockSpec((_KNN_BLK, _K + 1), lambda i: (i, 0)),
        ],
        out_shape=[
            jax.ShapeDtypeStruct((n, _K + 1), f32),
            jax.ShapeDtypeStruct((n, _K + 1), jnp.int32),
        ],
    )(z_n, z_n)

    mm_flat, xm_flat = _sc_sort(x.reshape(-1), x_mask.reshape(-1), comp.reshape(-1))
    x_mask_merge = mm_flat.reshape(n, 2 * l)
    x_merge = xm_flat.reshape(n, 2 * l)

    dist = vals[:, 1:]
    idx = idxs[:, 1:]
    idx_source = jnp.repeat(jnp.arange(n, dtype=jnp.int32), _K)
    edge_index = jnp.stack([idx_source, idx.reshape(-1)], axis=0)
    attr = dist.reshape(-1, 1)
    return (x_merge, edge_index, attr, y_merge2.reshape(n),
            temporal_edge_index, temporal_edge_attr, x_mask_merge)


# speedup vs baseline: 2.2538x; 1.1261x over previous
import functools

import jax
import jax.numpy as jnp
from jax import lax
from jax.experimental import pallas as pl
from jax.experimental.pallas import tpu as pltpu
from jax.experimental.pallas import tpu_sc as plsc

_HIDDEN = 512
_SUBSEQ = 200
_K = 5

_MLP_BLK = 512
_RANK_BLK = 256
_MERGE_BLK = 512
_KNN_BLK = 256

_N = 4096
_L = 400
_W = 2 * _L
_NW = 32
_RPW = _N // _NW
_BATCH = 16
_NB = _RPW // _BATCH
_NCH = _W // 16
_KMAX = 0x3F7FFFFF


def _mlp_kernel(z_ref, w1_ref, b1_ref, w2_ref, b2_ref, out_ref):
    h = jnp.dot(z_ref[...], w1_ref[...], preferred_element_type=jnp.float32)
    h = jnp.maximum(h + b1_ref[...], 0.0)
    out_ref[...] = jnp.dot(h, w2_ref[...], preferred_element_type=jnp.float32) + b2_ref[...]


def _zsim_kernel(n, blk, zp_ref, zr_ref, out_ref):
    i = pl.program_id(0)
    s = jnp.sum(zp_ref[...] * zr_ref[...], axis=1, keepdims=True)
    rows = i * blk + jax.lax.broadcasted_iota(jnp.int32, (blk, 1), 0)
    out_ref[...] = jnp.where(rows == n - 1, -100000000.0, s)


def _rank_kernel(n, blk, comp_count, scol_ref, srow_ref, out_ref):
    i = pl.program_id(0)
    sc = scol_ref[...]
    sr = srow_ref[...]
    jidx = jax.lax.broadcasted_iota(jnp.int32, (1, n), 1)
    iidx = i * blk + jax.lax.broadcasted_iota(jnp.int32, (blk, 1), 0)
    lt = (sr < sc).astype(jnp.float32)
    eqlt = ((sr == sc) & (jidx < iidx)).astype(jnp.float32)
    cnt = jnp.sum(lt + eqlt, axis=1, keepdims=True)
    out_ref[...] = (cnt < float(comp_count)).astype(jnp.float32)


def _merge_kernel(z_ref, zr_ref, y_ref, yr_ref, comp_ref, zn_ref, ym_ref):
    comp = comp_ref[...] > 0.5
    z = z_ref[...]
    zm = jnp.where(comp, z, (z + zr_ref[...]) / 2.0)
    nrm = jnp.sqrt(jnp.sum(zm * zm, axis=1, keepdims=True))
    zn_ref[...] = zm / jnp.maximum(nrm, 1e-12)
    y = y_ref[...]
    ym_ref[...] = jnp.where(comp, y, jnp.minimum(y, yr_ref[...]))


def _knn_kernel(n, blk, zi_ref, zall_ref, vals_ref, idxs_ref):
    a = jax.lax.dot_general(
        zi_ref[...], zall_ref[...], (((1,), (1,)), ((), ())),
        preferred_element_type=jnp.float32)
    lane = jax.lax.broadcasted_iota(jnp.int32, (blk, n), 1)
    vals, idxs = [], []
    for _ in range(_K + 1):
        m = jnp.max(a, axis=1, keepdims=True)
        am = jnp.min(jnp.where(a == m, lane, n), axis=1, keepdims=True)
        vals.append(m)
        idxs.append(am)
        a = jnp.where(lane == am, -jnp.inf, a)
    vals_ref[...] = jnp.concatenate(vals, axis=1)
    idxs_ref[...] = jnp.concatenate(idxs, axis=1)


_SC_MESH = plsc.VectorSubcoreMesh(core_axis_name="c", subcore_axis_name="s")


@functools.partial(
    pl.kernel, mesh=_SC_MESH,
    out_type=[
        jax.ShapeDtypeStruct((_N * _W,), jnp.float32),
        jax.ShapeDtypeStruct((_N * _W,), jnp.float32),
    ],
    scratch_types=[
        pltpu.VMEM(((_BATCH + 1) * _L,), jnp.float32),
        pltpu.VMEM(((_BATCH + 1) * _L,), jnp.float32),
        pltpu.VMEM((_BATCH * _W,), jnp.float32),
        pltpu.VMEM((_BATCH * _W,), jnp.float32),
        pltpu.VMEM((_W,), jnp.int32),
        pltpu.VMEM((_W,), jnp.float32),
        pltpu.VMEM((_W,), jnp.int32),
        pltpu.VMEM((_W,), jnp.float32),
        pltpu.VMEM((256,), jnp.int32),
        pltpu.VMEM((256,), jnp.int32),
        pltpu.VMEM((_RPW,), jnp.float32),
    ],
    compiler_params=pltpu.CompilerParams(needs_layout_passes=False),
)
def _sc_sort(x_hbm, m_hbm, comp_hbm, outk_hbm, outp_hbm,
             m_buf, x_buf, outk_buf, outp_buf, a_k, a_p, b_k, b_p, cnt, rr, comp_v):
    wid = lax.axis_index("s") * 2 + lax.axis_index("c")
    iota = jax.lax.iota(jnp.int32, 16)
    ones = jnp.ones(16, jnp.int32)
    zeros16 = jnp.zeros(16, jnp.int32)

    pltpu.sync_copy(comp_hbm.at[pl.ds(wid * _RPW, _RPW)], comp_v)

    def zero_cnt(i, c):
        cnt[pl.ds(i * 16, 16)] = zeros16
        return c

    def prefix(i, carry):
        v = cnt[pl.ds(i * 16, 16)]
        inc = plsc.cumsum(v)
        rr[pl.ds(i * 16, 16)] = inc - v + carry
        return carry + jnp.sum(v)

    def batch_body(rb, carry0):
        r0 = wid * _RPW + rb * _BATCH
        re = lax.rem(r0 + _BATCH, _N)
        pltpu.sync_copy(m_hbm.at[pl.ds(r0 * _L, _BATCH * _L)], m_buf.at[pl.ds(0, _BATCH * _L)])
        pltpu.sync_copy(m_hbm.at[pl.ds(re * _L, _L)], m_buf.at[pl.ds(_BATCH * _L, _L)])
        pltpu.sync_copy(x_hbm.at[pl.ds(r0 * _L, _BATCH * _L)], x_buf.at[pl.ds(0, _BATCH * _L)])
        pltpu.sync_copy(x_hbm.at[pl.ds(re * _L, _L)], x_buf.at[pl.ds(_BATCH * _L, _L)])

        def row_body(rl, rcarry):
            mo = rl * _L
            oo = rl * _W
            cv = plsc.load_gather(comp_v, [zeros16 + (rb * _BATCH + rl)])
            is_comp = cv > 0.5

            lax.fori_loop(0, 16, zero_cnt, 0)

            def count0_one(c):
                raw = m_buf[pl.ds(mo + c * 16, 16)]
                gl = c * 16 + iota
                zc = is_comp & (gl >= _W - _SUBSEQ)
                val = jnp.where(zc, 0.0, raw)
                tk = _KMAX - plsc.bitcast(val, jnp.int32)
                a_k[pl.ds(c * 16, 16)] = tk
                plsc.addupdate_scatter(cnt, [tk & 255], ones)

            def count0(i, cy):
                count0_one(2 * i)
                count0_one(2 * i + 1)
                return cy
            lax.fori_loop(0, _NCH // 2, count0, 0)

            def make_scat(sk, sp, dk, dp, shift, out_off, next_hist, back_transform):
                def one(c):
                    tk = sk[pl.ds(c * 16, 16)]
                    pv = sp[pl.ds(mo + c * 16, 16)] if sp is x_buf else sp[pl.ds(c * 16, 16)]
                    dig = lax.shift_right_logical(tk, shift) & 255 if shift else tk & 255
                    c1, lastm = plsc.scan_count(dig)
                    pos = out_off + plsc.load_gather(rr, [dig]) + c1 - 1
                    if back_transform:
                        plsc.store_scatter(dk, [pos], plsc.bitcast(_KMAX - tk, jnp.float32))
                    else:
                        plsc.store_scatter(dk, [pos], tk)
                    plsc.store_scatter(dp, [pos], pv)
                    plsc.addupdate_scatter(rr, [dig], c1, mask=lastm)
                    if next_hist:
                        plsc.addupdate_scatter(
                            cnt, [lax.shift_right_logical(tk, shift + 8) & 255], ones)

                def body(i, cy):
                    one(2 * i)
                    one(2 * i + 1)
                    return cy
                return body

            passes = [
                (a_k, x_buf, b_k, b_p, 0, 0, True, False),
                (b_k, b_p, a_k, a_p, 8, 0, True, False),
                (a_k, a_p, b_k, b_p, 16, 0, True, False),
                (b_k, b_p, outk_buf, outp_buf, 24, oo, False, True),
            ]
            for sk, sp, dk, dp, shift, out_off, nh, bt in passes:
                lax.fori_loop(0, 16, prefix, jnp.int32(0))
                lax.fori_loop(0, 16, zero_cnt, 0)
                lax.fori_loop(0, _NCH // 2,
                              make_scat(sk, sp, dk, dp, shift, out_off, nh, bt), 0)
            return rcarry

        lax.fori_loop(0, _BATCH, row_body, 0)
        pltpu.sync_copy(outk_buf, outk_hbm.at[pl.ds(r0 * _W, _BATCH * _W)])
        pltpu.sync_copy(outp_buf, outp_hbm.at[pl.ds(r0 * _W, _BATCH * _W)])
        return carry0

    lax.fori_loop(0, _NB, batch_body, 0)


def kernel(z, x, y, x_mask, temporal_edge_index, temporal_edge_attr,
           sliding_wdw, W1, b1, W2, b2):
    n, dd = z.shape
    l = x.shape[1]
    merge_num = n // 2
    f32 = jnp.float32

    z_proj = pl.pallas_call(
        _mlp_kernel,
        grid=(n // _MLP_BLK,),
        in_specs=[
            pl.BlockSpec((_MLP_BLK, dd), lambda i: (i, 0)),
            pl.BlockSpec((dd, _HIDDEN), lambda i: (0, 0)),
            pl.BlockSpec((1, _HIDDEN), lambda i: (0, 0)),
            pl.BlockSpec((_HIDDEN, _HIDDEN), lambda i: (0, 0)),
            pl.BlockSpec((1, _HIDDEN), lambda i: (0, 0)),
        ],
        out_specs=pl.BlockSpec((_MLP_BLK, _HIDDEN), lambda i: (i, 0)),
        out_shape=jax.ShapeDtypeStruct((n, _HIDDEN), f32),
    )(z, W1, b1.reshape(1, -1), W2, b2.reshape(1, -1))

    zp_roll = jnp.roll(z_proj, -1, axis=0)
    z_sim = pl.pallas_call(
        functools.partial(_zsim_kernel, n, _MLP_BLK),
        grid=(n // _MLP_BLK,),
        in_specs=[
            pl.BlockSpec((_MLP_BLK, _HIDDEN), lambda i: (i, 0)),
            pl.BlockSpec((_MLP_BLK, _HIDDEN), lambda i: (i, 0)),
        ],
        out_specs=pl.BlockSpec((_MLP_BLK, 1), lambda i: (i, 0)),
        out_shape=jax.ShapeDtypeStruct((n, 1), f32),
    )(z_proj, zp_roll)

    comp = pl.pallas_call(
        functools.partial(_rank_kernel, n, _RANK_BLK, n - merge_num),
        grid=(n // _RANK_BLK,),
        in_specs=[
            pl.BlockSpec((_RANK_BLK, 1), lambda i: (i, 0)),
            pl.BlockSpec((1, n), lambda i: (0, 0)),
        ],
        out_specs=pl.BlockSpec((_RANK_BLK, 1), lambda i: (i, 0)),
        out_shape=jax.ShapeDtypeStruct((n, 1), f32),
    )(z_sim, z_sim.reshape(1, n))

    z_roll = jnp.roll(z, -1, axis=0)
    y2 = y.reshape(n, 1)
    y2_roll = jnp.roll(y2, -1, axis=0)
    z_n, y_merge2 = pl.pallas_call(
        _merge_kernel,
        grid=(n // _MERGE_BLK,),
        in_specs=[
            pl.BlockSpec((_MERGE_BLK, dd), lambda i: (i, 0)),
            pl.BlockSpec((_MERGE_BLK, dd), lambda i: (i, 0)),
            pl.BlockSpec((_MERGE_BLK, 1), lambda i: (i, 0)),
            pl.BlockSpec((_MERGE_BLK, 1), lambda i: (i, 0)),
            pl.BlockSpec((_MERGE_BLK, 1), lambda i: (i, 0)),
        ],
        out_specs=[
            pl.BlockSpec((_MERGE_BLK, dd), lambda i: (i, 0)),
            pl.BlockSpec((_MERGE_BLK, 1), lambda i: (i, 0)),
        ],
        out_shape=[
            jax.ShapeDtypeStruct((n, dd), f32),
            jax.ShapeDtypeStruct((n, 1), f32),
        ],
    )(z, z_roll, y2, y2_roll, comp)

    vals, idxs = pl.pallas_call(
        functools.partial(_knn_kernel, n, _KNN_BLK),
        grid=(n // _KNN_BLK,),
        in_specs=[
            pl.BlockSpec((_KNN_BLK, dd), lambda i: (i, 0)),
            pl.BlockSpec((n, dd), lambda i: (0, 0)),
        ],
        out_specs=[
            pl.BlockSpec((_KNN_BLK, _K + 1), lambda i: (i, 0)),
            pl.BlockSpec((_KNN_BLK, _K + 1), lambda i: (i, 0)),
        ],
        out_shape=[
            jax.ShapeDtypeStruct((n, _K + 1), f32),
            jax.ShapeDtypeStruct((n, _K + 1), jnp.int32),
        ],
    )(z_n, z_n)

    mm_flat, xm_flat = _sc_sort(x.reshape(-1), x_mask.reshape(-1), comp.reshape(-1))
    x_mask_merge = mm_flat.reshape(n, 2 * l)
    x_merge = xm_flat.reshape(n, 2 * l)

    dist = vals[:, 1:]
    idx = idxs[:, 1:]
    idx_source = jnp.repeat(jnp.arange(n, dtype=jnp.int32), _K)
    edge_index = jnp.stack([idx_source, idx.reshape(-1)], axis=0)
    attr = dist.reshape(-1, 1)
    return (x_merge, edge_index, attr, y_merge2.reshape(n),
            temporal_edge_index, temporal_edge_attr, x_mask_merge)


# speedup vs baseline: 2.2545x; 1.0003x over previous
import functools

import jax
import jax.numpy as jnp
from jax import lax
from jax.experimental import pallas as pl
from jax.experimental.pallas import tpu as pltpu
from jax.experimental.pallas import tpu_sc as plsc

_HIDDEN = 512
_SUBSEQ = 200
_K = 5

_MLP_BLK = 512
_RANK_BLK = 256
_MERGE_BLK = 512
_KNN_BLK = 256

_N = 4096
_L = 400
_W = 2 * _L
_NW = 32
_RPW = _N // _NW
_BATCH = 16
_NB = _RPW // _BATCH
_NCH = _W // 16
_KMAX = 0x3F7FFFFF


def _mlp_kernel(z_ref, w1_ref, b1_ref, w2_ref, b2_ref, out_ref):
    h = jnp.dot(z_ref[...], w1_ref[...], preferred_element_type=jnp.float32)
    h = jnp.maximum(h + b1_ref[...], 0.0)
    out_ref[...] = jnp.dot(h, w2_ref[...], preferred_element_type=jnp.float32) + b2_ref[...]


def _zsim_kernel(n, blk, zp_ref, zr_ref, out_ref):
    i = pl.program_id(0)
    s = jnp.sum(zp_ref[...] * zr_ref[...], axis=1, keepdims=True)
    rows = i * blk + jax.lax.broadcasted_iota(jnp.int32, (blk, 1), 0)
    out_ref[...] = jnp.where(rows == n - 1, -100000000.0, s)


def _rank_kernel(n, blk, comp_count, scol_ref, srow_ref, out_ref):
    i = pl.program_id(0)
    sc = scol_ref[...]
    sr = srow_ref[...]
    jidx = jax.lax.broadcasted_iota(jnp.int32, (1, n), 1)
    iidx = i * blk + jax.lax.broadcasted_iota(jnp.int32, (blk, 1), 0)
    lt = (sr < sc).astype(jnp.float32)
    eqlt = ((sr == sc) & (jidx < iidx)).astype(jnp.float32)
    cnt = jnp.sum(lt + eqlt, axis=1, keepdims=True)
    out_ref[...] = (cnt < float(comp_count)).astype(jnp.float32)


def _merge_kernel(z_ref, zr_ref, y_ref, yr_ref, comp_ref, zn_ref, ym_ref):
    comp = comp_ref[...] > 0.5
    z = z_ref[...]
    zm = jnp.where(comp, z, (z + zr_ref[...]) / 2.0)
    nrm = jnp.sqrt(jnp.sum(zm * zm, axis=1, keepdims=True))
    zn_ref[...] = zm / jnp.maximum(nrm, 1e-12)
    y = y_ref[...]
    ym_ref[...] = jnp.where(comp, y, jnp.minimum(y, yr_ref[...]))


def _knn_kernel(n, blk, zi_ref, zall_ref, vals_ref, idxs_ref):
    a = jax.lax.dot_general(
        zi_ref[...], zall_ref[...], (((1,), (1,)), ((), ())),
        preferred_element_type=jnp.float32)
    lane = jax.lax.broadcasted_iota(jnp.int32, (blk, n), 1)
    vals, idxs = [], []
    for _ in range(_K + 1):
        m = jnp.max(a, axis=1, keepdims=True)
        am = jnp.min(jnp.where(a == m, lane, n), axis=1, keepdims=True)
        vals.append(m)
        idxs.append(am)
        a = jnp.where(lane == am, -jnp.inf, a)
    vals_ref[...] = jnp.concatenate(vals, axis=1)
    idxs_ref[...] = jnp.concatenate(idxs, axis=1)


_SC_MESH = plsc.VectorSubcoreMesh(core_axis_name="c", subcore_axis_name="s")


@functools.partial(
    pl.kernel, mesh=_SC_MESH,
    out_type=[
        jax.ShapeDtypeStruct((_N * _W,), jnp.float32),
        jax.ShapeDtypeStruct((_N * _W,), jnp.float32),
    ],
    scratch_types=[
        pltpu.VMEM(((_BATCH + 1) * _L,), jnp.float32),
        pltpu.VMEM(((_BATCH + 1) * _L,), jnp.float32),
        pltpu.VMEM((_BATCH * _W,), jnp.float32),
        pltpu.VMEM((_BATCH * _W,), jnp.float32),
        pltpu.VMEM((_W,), jnp.int32),
        pltpu.VMEM((_W,), jnp.float32),
        pltpu.VMEM((_W,), jnp.int32),
        pltpu.VMEM((_W,), jnp.float32),
        pltpu.VMEM((256,), jnp.int32),
        pltpu.VMEM((256,), jnp.int32),
        pltpu.VMEM((_RPW,), jnp.float32),
    ],
    compiler_params=pltpu.CompilerParams(needs_layout_passes=False),
    cost_estimate=pl.CostEstimate(
        flops=200_000_000, transcendentals=0, bytes_accessed=110_000_000),
)
def _sc_sort(x_hbm, m_hbm, comp_hbm, outk_hbm, outp_hbm,
             m_buf, x_buf, outk_buf, outp_buf, a_k, a_p, b_k, b_p, cnt, rr, comp_v):
    wid = lax.axis_index("s") * 2 + lax.axis_index("c")
    iota = jax.lax.iota(jnp.int32, 16)
    ones = jnp.ones(16, jnp.int32)
    zeros16 = jnp.zeros(16, jnp.int32)

    pltpu.sync_copy(comp_hbm.at[pl.ds(wid * _RPW, _RPW)], comp_v)

    def zero_cnt(i, c):
        cnt[pl.ds(i * 16, 16)] = zeros16
        return c

    def prefix(i, carry):
        v = cnt[pl.ds(i * 16, 16)]
        inc = plsc.cumsum(v)
        rr[pl.ds(i * 16, 16)] = inc - v + carry
        return carry + jnp.sum(v)

    def batch_body(rb, carry0):
        r0 = wid * _RPW + rb * _BATCH
        re = lax.rem(r0 + _BATCH, _N)
        pltpu.sync_copy(m_hbm.at[pl.ds(r0 * _L, _BATCH * _L)], m_buf.at[pl.ds(0, _BATCH * _L)])
        pltpu.sync_copy(m_hbm.at[pl.ds(re * _L, _L)], m_buf.at[pl.ds(_BATCH * _L, _L)])
        pltpu.sync_copy(x_hbm.at[pl.ds(r0 * _L, _BATCH * _L)], x_buf.at[pl.ds(0, _BATCH * _L)])
        pltpu.sync_copy(x_hbm.at[pl.ds(re * _L, _L)], x_buf.at[pl.ds(_BATCH * _L, _L)])

        def row_body(rl, rcarry):
            mo = rl * _L
            oo = rl * _W
            cv = plsc.load_gather(comp_v, [zeros16 + (rb * _BATCH + rl)])
            is_comp = cv > 0.5

            lax.fori_loop(0, 16, zero_cnt, 0)

            def count0_one(c):
                raw = m_buf[pl.ds(mo + c * 16, 16)]
                gl = c * 16 + iota
                zc = is_comp & (gl >= _W - _SUBSEQ)
                val = jnp.where(zc, 0.0, raw)
                tk = _KMAX - plsc.bitcast(val, jnp.int32)
                a_k[pl.ds(c * 16, 16)] = tk
                plsc.addupdate_scatter(cnt, [tk & 255], ones)

            def count0(i, cy):
                count0_one(2 * i)
                count0_one(2 * i + 1)
                return cy
            lax.fori_loop(0, _NCH // 2, count0, 0)

            def make_scat(sk, sp, dk, dp, shift, out_off, next_hist, back_transform):
                def one(c):
                    tk = sk[pl.ds(c * 16, 16)]
                    pv = sp[pl.ds(mo + c * 16, 16)] if sp is x_buf else sp[pl.ds(c * 16, 16)]
                    dig = lax.shift_right_logical(tk, shift) & 255 if shift else tk & 255
                    c1, lastm = plsc.scan_count(dig)
                    pos = out_off + plsc.load_gather(rr, [dig]) + c1 - 1
                    if back_transform:
                        plsc.store_scatter(dk, [pos], plsc.bitcast(_KMAX - tk, jnp.float32))
                    else:
                        plsc.store_scatter(dk, [pos], tk)
                    plsc.store_scatter(dp, [pos], pv)
                    plsc.addupdate_scatter(rr, [dig], c1, mask=lastm)
                    if next_hist:
                        plsc.addupdate_scatter(
                            cnt, [lax.shift_right_logical(tk, shift + 8) & 255], ones)

                def body(i, cy):
                    one(2 * i)
                    one(2 * i + 1)
                    return cy
                return body

            passes = [
                (a_k, x_buf, b_k, b_p, 0, 0, True, False),
                (b_k, b_p, a_k, a_p, 8, 0, True, False),
                (a_k, a_p, b_k, b_p, 16, 0, True, False),
                (b_k, b_p, outk_buf, outp_buf, 24, oo, False, True),
            ]
            for sk, sp, dk, dp, shift, out_off, nh, bt in passes:
                lax.fori_loop(0, 16, prefix, jnp.int32(0))
                lax.fori_loop(0, 16, zero_cnt, 0)
                lax.fori_loop(0, _NCH // 2,
                              make_scat(sk, sp, dk, dp, shift, out_off, nh, bt), 0)
            return rcarry

        lax.fori_loop(0, _BATCH, row_body, 0)
        pltpu.sync_copy(outk_buf, outk_hbm.at[pl.ds(r0 * _W, _BATCH * _W)])
        pltpu.sync_copy(outp_buf, outp_hbm.at[pl.ds(r0 * _W, _BATCH * _W)])
        return carry0

    lax.fori_loop(0, _NB, batch_body, 0)


def kernel(z, x, y, x_mask, temporal_edge_index, temporal_edge_attr,
           sliding_wdw, W1, b1, W2, b2):
    n, dd = z.shape
    l = x.shape[1]
    merge_num = n // 2
    f32 = jnp.float32

    z_proj = pl.pallas_call(
        _mlp_kernel,
        grid=(n // _MLP_BLK,),
        in_specs=[
            pl.BlockSpec((_MLP_BLK, dd), lambda i: (i, 0)),
            pl.BlockSpec((dd, _HIDDEN), lambda i: (0, 0)),
            pl.BlockSpec((1, _HIDDEN), lambda i: (0, 0)),
            pl.BlockSpec((_HIDDEN, _HIDDEN), lambda i: (0, 0)),
            pl.BlockSpec((1, _HIDDEN), lambda i: (0, 0)),
        ],
        out_specs=pl.BlockSpec((_MLP_BLK, _HIDDEN), lambda i: (i, 0)),
        out_shape=jax.ShapeDtypeStruct((n, _HIDDEN), f32),
    )(z, W1, b1.reshape(1, -1), W2, b2.reshape(1, -1))

    zp_roll = jnp.roll(z_proj, -1, axis=0)
    z_sim = pl.pallas_call(
        functools.partial(_zsim_kernel, n, _MLP_BLK),
        grid=(n // _MLP_BLK,),
        in_specs=[
            pl.BlockSpec((_MLP_BLK, _HIDDEN), lambda i: (i, 0)),
            pl.BlockSpec((_MLP_BLK, _HIDDEN), lambda i: (i, 0)),
        ],
        out_specs=pl.BlockSpec((_MLP_BLK, 1), lambda i: (i, 0)),
        out_shape=jax.ShapeDtypeStruct((n, 1), f32),
    )(z_proj, zp_roll)

    comp = pl.pallas_call(
        functools.partial(_rank_kernel, n, _RANK_BLK, n - merge_num),
        grid=(n // _RANK_BLK,),
        in_specs=[
            pl.BlockSpec((_RANK_BLK, 1), lambda i: (i, 0)),
            pl.BlockSpec((1, n), lambda i: (0, 0)),
        ],
        out_specs=pl.BlockSpec((_RANK_BLK, 1), lambda i: (i, 0)),
        out_shape=jax.ShapeDtypeStruct((n, 1), f32),
    )(z_sim, z_sim.reshape(1, n))

    z_roll = jnp.roll(z, -1, axis=0)
    y2 = y.reshape(n, 1)
    y2_roll = jnp.roll(y2, -1, axis=0)
    z_n, y_merge2 = pl.pallas_call(
        _merge_kernel,
        grid=(n // _MERGE_BLK,),
        in_specs=[
            pl.BlockSpec((_MERGE_BLK, dd), lambda i: (i, 0)),
            pl.BlockSpec((_MERGE_BLK, dd), lambda i: (i, 0)),
            pl.BlockSpec((_MERGE_BLK, 1), lambda i: (i, 0)),
            pl.BlockSpec((_MERGE_BLK, 1), lambda i: (i, 0)),
            pl.BlockSpec((_MERGE_BLK, 1), lambda i: (i, 0)),
        ],
        out_specs=[
            pl.BlockSpec((_MERGE_BLK, dd), lambda i: (i, 0)),
            pl.BlockSpec((_MERGE_BLK, 1), lambda i: (i, 0)),
        ],
        out_shape=[
            jax.ShapeDtypeStruct((n, dd), f32),
            jax.ShapeDtypeStruct((n, 1), f32),
        ],
    )(z, z_roll, y2, y2_roll, comp)

    vals, idxs = pl.pallas_call(
        functools.partial(_knn_kernel, n, _KNN_BLK),
        grid=(n // _KNN_BLK,),
        in_specs=[
            pl.BlockSpec((_KNN_BLK, dd), lambda i: (i, 0)),
            pl.BlockSpec((n, dd), lambda i: (0, 0)),
        ],
        out_specs=[
            pl.BlockSpec((_KNN_BLK, _K + 1), lambda i: (i, 0)),
            pl.BlockSpec((_KNN_BLK, _K + 1), lambda i: (i, 0)),
        ],
        out_shape=[
            jax.ShapeDtypeStruct((n, _K + 1), f32),
            jax.ShapeDtypeStruct((n, _K + 1), jnp.int32),
        ],
    )(z_n, z_n)

    mm_flat, xm_flat = _sc_sort(x.reshape(-1), x_mask.reshape(-1), comp.reshape(-1))
    x_mask_merge = mm_flat.reshape(n, 2 * l)
    x_merge = xm_flat.reshape(n, 2 * l)

    dist = vals[:, 1:]
    idx = idxs[:, 1:]
    idx_source = jnp.repeat(jnp.arange(n, dtype=jnp.int32), _K)
    edge_index = jnp.stack([idx_source, idx.reshape(-1)], axis=0)
    attr = dist.reshape(-1, 1)
    return (x_merge, edge_index, attr, y_merge2.reshape(n),
            temporal_edge_index, temporal_edge_attr, x_mask_merge)


# speedup vs baseline: 2.3145x; 1.0266x over previous
import functools

import jax
import jax.numpy as jnp
from jax import lax
from jax.experimental import pallas as pl
from jax.experimental.pallas import tpu as pltpu
from jax.experimental.pallas import tpu_sc as plsc

_HIDDEN = 512
_SUBSEQ = 200
_K = 5

_MLP_BLK = 512
_RANK_BLK = 256
_MERGE_BLK = 512
_KNN_BLK = 256

_N = 4096
_L = 400
_W = 2 * _L
_NW = 32
_RPW = _N // _NW
_BATCH = 16
_NB = _RPW // _BATCH
_NCH = _W // 16
_KMAX = 0x3F7FFFFF


def _mlp_kernel(z_ref, w1_ref, b1_ref, w2_ref, b2_ref, out_ref):
    h = jnp.dot(z_ref[...], w1_ref[...], preferred_element_type=jnp.float32)
    h = jnp.maximum(h + b1_ref[...], 0.0)
    out_ref[...] = jnp.dot(h, w2_ref[...], preferred_element_type=jnp.float32) + b2_ref[...]


def _zsim_kernel(n, blk, zp_ref, zr_ref, out_ref):
    i = pl.program_id(0)
    s = jnp.sum(zp_ref[...] * zr_ref[...], axis=1, keepdims=True)
    rows = i * blk + jax.lax.broadcasted_iota(jnp.int32, (blk, 1), 0)
    out_ref[...] = jnp.where(rows == n - 1, -100000000.0, s)


def _rank_kernel(n, blk, comp_count, scol_ref, srow_ref, out_ref):
    i = pl.program_id(0)
    sc = scol_ref[...]
    sr = srow_ref[...]
    jidx = jax.lax.broadcasted_iota(jnp.int32, (1, n), 1)
    iidx = i * blk + jax.lax.broadcasted_iota(jnp.int32, (blk, 1), 0)
    lt = (sr < sc).astype(jnp.float32)
    eqlt = ((sr == sc) & (jidx < iidx)).astype(jnp.float32)
    cnt = jnp.sum(lt + eqlt, axis=1, keepdims=True)
    out_ref[...] = (cnt < float(comp_count)).astype(jnp.float32)


def _merge_kernel(z_ref, zr_ref, y_ref, yr_ref, comp_ref, zn_ref, ym_ref):
    comp = comp_ref[...] > 0.5
    z = z_ref[...]
    zm = jnp.where(comp, z, (z + zr_ref[...]) / 2.0)
    nrm = jnp.sqrt(jnp.sum(zm * zm, axis=1, keepdims=True))
    zn_ref[...] = zm / jnp.maximum(nrm, 1e-12)
    y = y_ref[...]
    ym_ref[...] = jnp.where(comp, y, jnp.minimum(y, yr_ref[...]))


def _knn_kernel(n, blk, zi_ref, zall_ref, vals_ref, idxs_ref):
    a = jax.lax.dot_general(
        zi_ref[...], zall_ref[...], (((1,), (1,)), ((), ())),
        preferred_element_type=jnp.float32)
    lane = jax.lax.broadcasted_iota(jnp.int32, (blk, n), 1)
    vals, idxs = [], []
    for _ in range(_K + 1):
        m = jnp.max(a, axis=1, keepdims=True)
        am = jnp.min(jnp.where(a == m, lane, n), axis=1, keepdims=True)
        vals.append(m)
        idxs.append(am)
        a = jnp.where(lane == am, -jnp.inf, a)
    vals_ref[...] = jnp.concatenate(vals, axis=1)
    idxs_ref[...] = jnp.concatenate(idxs, axis=1)


_SC_MESH = plsc.VectorSubcoreMesh(core_axis_name="c", subcore_axis_name="s")


@functools.partial(
    pl.kernel, mesh=_SC_MESH,
    out_type=[
        jax.ShapeDtypeStruct((_N * _W,), jnp.float32),
        jax.ShapeDtypeStruct((_N * _W,), jnp.float32),
    ],
    scratch_types=[
        pltpu.VMEM(((_BATCH + 1) * _L,), jnp.float32),
        pltpu.VMEM(((_BATCH + 1) * _L,), jnp.float32),
        pltpu.VMEM((_BATCH * _W,), jnp.float32),
        pltpu.VMEM((_BATCH * _W,), jnp.float32),
        pltpu.VMEM((_W,), jnp.int32),
        pltpu.VMEM((_W,), jnp.float32),
        pltpu.VMEM((_W,), jnp.int32),
        pltpu.VMEM((_W,), jnp.float32),
        pltpu.VMEM((256,), jnp.int32),
        pltpu.VMEM((256,), jnp.int32),
        pltpu.VMEM((_W,), jnp.int32),
        pltpu.VMEM((_W,), jnp.float32),
        pltpu.VMEM((_W,), jnp.int32),
        pltpu.VMEM((_W,), jnp.float32),
        pltpu.VMEM((256,), jnp.int32),
        pltpu.VMEM((256,), jnp.int32),
        pltpu.VMEM((_RPW,), jnp.float32),
    ],
    compiler_params=pltpu.CompilerParams(needs_layout_passes=False),
    cost_estimate=pl.CostEstimate(
        flops=200_000_000, transcendentals=0, bytes_accessed=110_000_000),
)
def _sc_sort(x_hbm, m_hbm, comp_hbm, outk_hbm, outp_hbm,
             m_buf, x_buf, outk_buf, outp_buf,
             a_k0, a_p0, b_k0, b_p0, cnt0, rr0,
             a_k1, a_p1, b_k1, b_p1, cnt1, rr1, comp_v):
    wid = lax.axis_index("s") * 2 + lax.axis_index("c")
    iota = jax.lax.iota(jnp.int32, 16)
    ones = jnp.ones(16, jnp.int32)
    zeros16 = jnp.zeros(16, jnp.int32)

    pltpu.sync_copy(comp_hbm.at[pl.ds(wid * _RPW, _RPW)], comp_v)

    LANES = ((a_k0, a_p0, b_k0, b_p0, cnt0, rr0),
             (a_k1, a_p1, b_k1, b_p1, cnt1, rr1))

    def zero_cnt(i, c):
        cnt0[pl.ds(i * 16, 16)] = zeros16
        cnt1[pl.ds(i * 16, 16)] = zeros16
        return c

    def prefix(i, carry):
        ca, cb = carry
        v0 = cnt0[pl.ds(i * 16, 16)]
        v1 = cnt1[pl.ds(i * 16, 16)]
        inc0 = plsc.cumsum(v0)
        inc1 = plsc.cumsum(v1)
        rr0[pl.ds(i * 16, 16)] = inc0 - v0 + ca
        rr1[pl.ds(i * 16, 16)] = inc1 - v1 + cb
        return (ca + jnp.sum(v0), cb + jnp.sum(v1))

    def batch_body(rb, carry0):
        r0 = wid * _RPW + rb * _BATCH
        re = lax.rem(r0 + _BATCH, _N)
        pltpu.sync_copy(m_hbm.at[pl.ds(r0 * _L, _BATCH * _L)], m_buf.at[pl.ds(0, _BATCH * _L)])
        pltpu.sync_copy(m_hbm.at[pl.ds(re * _L, _L)], m_buf.at[pl.ds(_BATCH * _L, _L)])
        pltpu.sync_copy(x_hbm.at[pl.ds(r0 * _L, _BATCH * _L)], x_buf.at[pl.ds(0, _BATCH * _L)])
        pltpu.sync_copy(x_hbm.at[pl.ds(re * _L, _L)], x_buf.at[pl.ds(_BATCH * _L, _L)])

        def pair_body(rp, rcarry):
            rl0 = 2 * rp
            mos = (rl0 * _L, (rl0 + 1) * _L)
            oos = (rl0 * _W, (rl0 + 1) * _W)
            comps = []
            for j in (0, 1):
                cv = plsc.load_gather(comp_v, [zeros16 + (rb * _BATCH + rl0 + j)])
                comps.append(cv > 0.5)

            lax.fori_loop(0, 16, zero_cnt, 0)

            def count0(c, cy):
                for j in (0, 1):
                    a_k, _, _, _, cnt, _ = LANES[j]
                    raw = m_buf[pl.ds(mos[j] + c * 16, 16)]
                    gl = c * 16 + iota
                    zc = comps[j] & (gl >= _W - _SUBSEQ)
                    val = jnp.where(zc, 0.0, raw)
                    tk = _KMAX - plsc.bitcast(val, jnp.int32)
                    a_k[pl.ds(c * 16, 16)] = tk
                    plsc.addupdate_scatter(cnt, [tk & 255], ones)
                return cy
            lax.fori_loop(0, _NCH, count0, 0)

            def make_scat(pno):
                shift = 8 * pno
                last = pno == 3

                def body(c, cy):
                    for j in (0, 1):
                        a_k, a_p, b_k, b_p, cnt, rr = LANES[j]
                        if pno == 0:
                            sk, sp_load = a_k, lambda jj=j: x_buf[pl.ds(mos[jj] + c * 16, 16)]
                            dk, dp = b_k, b_p
                        elif pno == 1:
                            sk, sp_load = b_k, lambda b_p=b_p: b_p[pl.ds(c * 16, 16)]
                            dk, dp = a_k, a_p
                        elif pno == 2:
                            sk, sp_load = a_k, lambda a_p=a_p: a_p[pl.ds(c * 16, 16)]
                            dk, dp = b_k, b_p
                        else:
                            sk, sp_load = b_k, lambda b_p=b_p: b_p[pl.ds(c * 16, 16)]
                            dk, dp = outk_buf, outp_buf
                        tk = sk[pl.ds(c * 16, 16)]
                        pv = sp_load()
                        dig = lax.shift_right_logical(tk, shift) & 255 if shift else tk & 255
                        c1, lastm = plsc.scan_count(dig)
                        pos = plsc.load_gather(rr, [dig]) + c1 - 1
                        if last:
                            pos = pos + oos[j]
                            plsc.store_scatter(dk, [pos], plsc.bitcast(_KMAX - tk, jnp.float32))
                        else:
                            plsc.store_scatter(dk, [pos], tk)
                        plsc.store_scatter(dp, [pos], pv)
                        plsc.addupdate_scatter(rr, [dig], c1, mask=lastm)
                        if not last:
                            plsc.addupdate_scatter(
                                cnt, [lax.shift_right_logical(tk, shift + 8) & 255], ones)
                    return cy
                return body

            for pno in range(4):
                lax.fori_loop(0, 16, prefix, (jnp.int32(0), jnp.int32(0)))
                lax.fori_loop(0, 16, zero_cnt, 0)
                lax.fori_loop(0, _NCH, make_scat(pno), 0)
            return rcarry

        lax.fori_loop(0, _BATCH // 2, pair_body, 0)
        pltpu.sync_copy(outk_buf, outk_hbm.at[pl.ds(r0 * _W, _BATCH * _W)])
        pltpu.sync_copy(outp_buf, outp_hbm.at[pl.ds(r0 * _W, _BATCH * _W)])
        return carry0

    lax.fori_loop(0, _NB, batch_body, 0)


def kernel(z, x, y, x_mask, temporal_edge_index, temporal_edge_attr,
           sliding_wdw, W1, b1, W2, b2):
    n, dd = z.shape
    l = x.shape[1]
    merge_num = n // 2
    f32 = jnp.float32

    z_proj = pl.pallas_call(
        _mlp_kernel,
        grid=(n // _MLP_BLK,),
        in_specs=[
            pl.BlockSpec((_MLP_BLK, dd), lambda i: (i, 0)),
            pl.BlockSpec((dd, _HIDDEN), lambda i: (0, 0)),
            pl.BlockSpec((1, _HIDDEN), lambda i: (0, 0)),
            pl.BlockSpec((_HIDDEN, _HIDDEN), lambda i: (0, 0)),
            pl.BlockSpec((1, _HIDDEN), lambda i: (0, 0)),
        ],
        out_specs=pl.BlockSpec((_MLP_BLK, _HIDDEN), lambda i: (i, 0)),
        out_shape=jax.ShapeDtypeStruct((n, _HIDDEN), f32),
    )(z, W1, b1.reshape(1, -1), W2, b2.reshape(1, -1))

    zp_roll = jnp.roll(z_proj, -1, axis=0)
    z_sim = pl.pallas_call(
        functools.partial(_zsim_kernel, n, _MLP_BLK),
        grid=(n // _MLP_BLK,),
        in_specs=[
            pl.BlockSpec((_MLP_BLK, _HIDDEN), lambda i: (i, 0)),
            pl.BlockSpec((_MLP_BLK, _HIDDEN), lambda i: (i, 0)),
        ],
        out_specs=pl.BlockSpec((_MLP_BLK, 1), lambda i: (i, 0)),
        out_shape=jax.ShapeDtypeStruct((n, 1), f32),
    )(z_proj, zp_roll)

    comp = pl.pallas_call(
        functools.partial(_rank_kernel, n, _RANK_BLK, n - merge_num),
        grid=(n // _RANK_BLK,),
        in_specs=[
            pl.BlockSpec((_RANK_BLK, 1), lambda i: (i, 0)),
            pl.BlockSpec((1, n), lambda i: (0, 0)),
        ],
        out_specs=pl.BlockSpec((_RANK_BLK, 1), lambda i: (i, 0)),
        out_shape=jax.ShapeDtypeStruct((n, 1), f32),
    )(z_sim, z_sim.reshape(1, n))

    z_roll = jnp.roll(z, -1, axis=0)
    y2 = y.reshape(n, 1)
    y2_roll = jnp.roll(y2, -1, axis=0)
    z_n, y_merge2 = pl.pallas_call(
        _merge_kernel,
        grid=(n // _MERGE_BLK,),
        in_specs=[
            pl.BlockSpec((_MERGE_BLK, dd), lambda i: (i, 0)),
            pl.BlockSpec((_MERGE_BLK, dd), lambda i: (i, 0)),
            pl.BlockSpec((_MERGE_BLK, 1), lambda i: (i, 0)),
            pl.BlockSpec((_MERGE_BLK, 1), lambda i: (i, 0)),
            pl.BlockSpec((_MERGE_BLK, 1), lambda i: (i, 0)),
        ],
        out_specs=[
            pl.BlockSpec((_MERGE_BLK, dd), lambda i: (i, 0)),
            pl.BlockSpec((_MERGE_BLK, 1), lambda i: (i, 0)),
        ],
        out_shape=[
            jax.ShapeDtypeStruct((n, dd), f32),
            jax.ShapeDtypeStruct((n, 1), f32),
        ],
    )(z, z_roll, y2, y2_roll, comp)

    vals, idxs = pl.pallas_call(
        functools.partial(_knn_kernel, n, _KNN_BLK),
        grid=(n // _KNN_BLK,),
        in_specs=[
            pl.BlockSpec((_KNN_BLK, dd), lambda i: (i, 0)),
            pl.BlockSpec((n, dd), lambda i: (0, 0)),
        ],
        out_specs=[
            pl.BlockSpec((_KNN_BLK, _K + 1), lambda i: (i, 0)),
            pl.BlockSpec((_KNN_BLK, _K + 1), lambda i: (i, 0)),
        ],
        out_shape=[
            jax.ShapeDtypeStruct((n, _K + 1), f32),
            jax.ShapeDtypeStruct((n, _K + 1), jnp.int32),
        ],
    )(z_n, z_n)

    mm_flat, xm_flat = _sc_sort(x.reshape(-1), x_mask.reshape(-1), comp.reshape(-1))
    x_mask_merge = mm_flat.reshape(n, 2 * l)
    x_merge = xm_flat.reshape(n, 2 * l)

    dist = vals[:, 1:]
    idx = idxs[:, 1:]
    idx_source = jnp.repeat(jnp.arange(n, dtype=jnp.int32), _K)
    edge_index = jnp.stack([idx_source, idx.reshape(-1)], axis=0)
    attr = dist.reshape(-1, 1)
    return (x_merge, edge_index, attr, y_merge2.reshape(n),
            temporal_edge_index, temporal_edge_attr, x_mask_merge)


# speedup vs baseline: 2.9157x; 1.2598x over previous
import functools

import jax
import jax.numpy as jnp
from jax import lax
from jax.experimental import pallas as pl
from jax.experimental.pallas import tpu as pltpu
from jax.experimental.pallas import tpu_sc as plsc

_HIDDEN = 512
_SUBSEQ = 200
_K = 5

_MLP_BLK = 512
_RANK_BLK = 256
_MERGE_BLK = 512
_KNN_BLK = 256

_N = 4096
_L = 400
_W = 2 * _L
_NW = 32
_RPW = _N // _NW
_BATCH = 16
_NB = _RPW // _BATCH
_NCH = _W // 16
_KMAX = 0x3F7FFFFF


def _mlp_kernel(z_ref, w1_ref, b1_ref, w2_ref, b2_ref, out_ref):
    h = jnp.dot(z_ref[...], w1_ref[...], preferred_element_type=jnp.float32)
    h = jnp.maximum(h + b1_ref[...], 0.0)
    out_ref[...] = jnp.dot(h, w2_ref[...], preferred_element_type=jnp.float32) + b2_ref[...]


def _zsim_kernel(n, blk, zp_ref, zr_ref, out_ref):
    i = pl.program_id(0)
    s = jnp.sum(zp_ref[...] * zr_ref[...], axis=1, keepdims=True)
    rows = i * blk + jax.lax.broadcasted_iota(jnp.int32, (blk, 1), 0)
    out_ref[...] = jnp.where(rows == n - 1, -100000000.0, s)


def _rank_kernel(n, blk, comp_count, scol_ref, srow_ref, out_ref):
    i = pl.program_id(0)
    sc = scol_ref[...]
    sr = srow_ref[...]
    jidx = jax.lax.broadcasted_iota(jnp.int32, (1, n), 1)
    iidx = i * blk + jax.lax.broadcasted_iota(jnp.int32, (blk, 1), 0)
    lt = (sr < sc).astype(jnp.float32)
    eqlt = ((sr == sc) & (jidx < iidx)).astype(jnp.float32)
    cnt = jnp.sum(lt + eqlt, axis=1, keepdims=True)
    out_ref[...] = (cnt < float(comp_count)).astype(jnp.float32)


def _merge_kernel(z_ref, zr_ref, y_ref, yr_ref, comp_ref, zn_ref, ym_ref):
    comp = comp_ref[...] > 0.5
    z = z_ref[...]
    zm = jnp.where(comp, z, (z + zr_ref[...]) / 2.0)
    nrm = jnp.sqrt(jnp.sum(zm * zm, axis=1, keepdims=True))
    zn_ref[...] = zm / jnp.maximum(nrm, 1e-12)
    y = y_ref[...]
    ym_ref[...] = jnp.where(comp, y, jnp.minimum(y, yr_ref[...]))


def _knn_kernel(n, blk, zi_ref, zall_ref, vals_ref, idxs_ref):
    a = jax.lax.dot_general(
        zi_ref[...], zall_ref[...], (((1,), (1,)), ((), ())),
        preferred_element_type=jnp.float32)
    lane = jax.lax.broadcasted_iota(jnp.int32, (blk, n), 1)
    vals, idxs = [], []
    for _ in range(_K + 1):
        m = jnp.max(a, axis=1, keepdims=True)
        am = jnp.min(jnp.where(a == m, lane, n), axis=1, keepdims=True)
        vals.append(m)
        idxs.append(am)
        a = jnp.where(lane == am, -jnp.inf, a)
    vals_ref[...] = jnp.concatenate(vals, axis=1)
    idxs_ref[...] = jnp.concatenate(idxs, axis=1)


_SC_MESH = plsc.VectorSubcoreMesh(core_axis_name="c", subcore_axis_name="s")


@functools.partial(
    pl.kernel, mesh=_SC_MESH,
    out_type=[
        jax.ShapeDtypeStruct((_N * _W,), jnp.float32),
        jax.ShapeDtypeStruct((_N * _W,), jnp.float32),
    ],
    scratch_types=[
        pltpu.VMEM(((_BATCH + 1) * _L,), jnp.float32),
        pltpu.VMEM(((_BATCH + 1) * _L,), jnp.float32),
        pltpu.VMEM((_BATCH * _W,), jnp.float32),
        pltpu.VMEM((_BATCH * _W,), jnp.float32),
        pltpu.VMEM((_W,), jnp.int32),
        pltpu.VMEM((_W,), jnp.float32),
        pltpu.VMEM((_W,), jnp.int32),
        pltpu.VMEM((_W,), jnp.float32),
        pltpu.VMEM((256,), jnp.int32),
        pltpu.VMEM((256,), jnp.int32),
        pltpu.VMEM((_W,), jnp.int32),
        pltpu.VMEM((_W,), jnp.float32),
        pltpu.VMEM((_W,), jnp.int32),
        pltpu.VMEM((_W,), jnp.float32),
        pltpu.VMEM((256,), jnp.int32),
        pltpu.VMEM((256,), jnp.int32),
        pltpu.VMEM((_RPW,), jnp.float32),
    ],
    compiler_params=pltpu.CompilerParams(needs_layout_passes=False),
    cost_estimate=pl.CostEstimate(
        flops=200_000_000, transcendentals=0, bytes_accessed=110_000_000),
)
def _sc_sort(x_hbm, m_hbm, comp_hbm, outk_hbm, outp_hbm,
             m_buf, x_buf, outk_buf, outp_buf,
             a_k0, a_p0, b_k0, b_p0, cnt0, rr0,
             a_k1, a_p1, b_k1, b_p1, cnt1, rr1, comp_v):
    wid = lax.axis_index("s") * 2 + lax.axis_index("c")
    iota = jax.lax.iota(jnp.int32, 16)
    ones = jnp.ones(16, jnp.int32)
    zeros16 = jnp.zeros(16, jnp.int32)

    pltpu.sync_copy(comp_hbm.at[pl.ds(wid * _RPW, _RPW)], comp_v)

    LANES = ((a_k0, a_p0, b_k0, b_p0, cnt0, rr0),
             (a_k1, a_p1, b_k1, b_p1, cnt1, rr1))

    def zero_cnt(i, c):
        cnt0[pl.ds(i * 16, 16)] = zeros16
        cnt1[pl.ds(i * 16, 16)] = zeros16
        return c

    def prefix(i, carry):
        ca, cb = carry
        v0 = cnt0[pl.ds(i * 16, 16)]
        v1 = cnt1[pl.ds(i * 16, 16)]
        inc0 = plsc.cumsum(v0)
        inc1 = plsc.cumsum(v1)
        rr0[pl.ds(i * 16, 16)] = inc0 - v0 + ca
        rr1[pl.ds(i * 16, 16)] = inc1 - v1 + cb
        return (ca + jnp.sum(v0), cb + jnp.sum(v1))

    def batch_body(rb, carry0):
        r0 = wid * _RPW + rb * _BATCH
        re = lax.rem(r0 + _BATCH, _N)
        pltpu.sync_copy(m_hbm.at[pl.ds(r0 * _L, _BATCH * _L)], m_buf.at[pl.ds(0, _BATCH * _L)])
        pltpu.sync_copy(m_hbm.at[pl.ds(re * _L, _L)], m_buf.at[pl.ds(_BATCH * _L, _L)])
        pltpu.sync_copy(x_hbm.at[pl.ds(r0 * _L, _BATCH * _L)], x_buf.at[pl.ds(0, _BATCH * _L)])
        pltpu.sync_copy(x_hbm.at[pl.ds(re * _L, _L)], x_buf.at[pl.ds(_BATCH * _L, _L)])

        def pair_body(rp, rcarry):
            rl0 = 2 * rp
            mos = (rl0 * _L, (rl0 + 1) * _L)
            oos = (rl0 * _W, (rl0 + 1) * _W)
            comps = []
            for j in (0, 1):
                cv = plsc.load_gather(comp_v, [zeros16 + (rb * _BATCH + rl0 + j)])
                comps.append(cv > 0.5)

            lax.fori_loop(0, 16, zero_cnt, 0)

            def count0(c, cy):
                for j in (0, 1):
                    a_k, _, _, _, cnt, _ = LANES[j]
                    raw = m_buf[pl.ds(mos[j] + c * 16, 16)]
                    gl = c * 16 + iota
                    zc = comps[j] & (gl >= _W - _SUBSEQ)
                    val = jnp.where(zc, 0.0, raw)
                    tk = _KMAX - plsc.bitcast(val, jnp.int32)
                    a_k[pl.ds(c * 16, 16)] = tk
                    plsc.addupdate_scatter(cnt, [tk & 255], ones)
                return cy
            lax.fori_loop(0, _NCH, count0, 0)

            def run_scat(pno):
                shift = 8 * pno
                last = pno == 3

                def fetch(j, c):
                    a_k, a_p, b_k, b_p, _, _ = LANES[j]
                    if pno == 0:
                        tk = a_k[pl.ds(c * 16, 16)]
                        pv = x_buf[pl.ds(mos[j] + c * 16, 16)]
                    elif pno == 1:
                        tk = b_k[pl.ds(c * 16, 16)]
                        pv = b_p[pl.ds(c * 16, 16)]
                    elif pno == 2:
                        tk = a_k[pl.ds(c * 16, 16)]
                        pv = a_p[pl.ds(c * 16, 16)]
                    else:
                        tk = b_k[pl.ds(c * 16, 16)]
                        pv = b_p[pl.ds(c * 16, 16)]
                    dig = lax.shift_right_logical(tk, shift) & 255 if shift else tk & 255
                    c1, lastm = plsc.scan_count(dig)
                    return tk, pv, dig, c1, lastm

                def emit(j, cur, extra_off):
                    a_k, a_p, b_k, b_p, cnt, rr = LANES[j]
                    if pno == 0:
                        dk, dp = b_k, b_p
                    elif pno == 1:
                        dk, dp = a_k, a_p
                    elif pno == 2:
                        dk, dp = b_k, b_p
                    else:
                        dk, dp = outk_buf, outp_buf
                    tk, pv, dig, c1, lastm = cur
                    pos = plsc.load_gather(rr, [dig]) + c1 - 1
                    if last:
                        pos = pos + extra_off
                        plsc.store_scatter(dk, [pos], plsc.bitcast(_KMAX - tk, jnp.float32))
                    else:
                        plsc.store_scatter(dk, [pos], tk)
                    plsc.store_scatter(dp, [pos], pv)
                    plsc.addupdate_scatter(rr, [dig], c1, mask=lastm)
                    if not last:
                        plsc.addupdate_scatter(
                            cnt, [lax.shift_right_logical(tk, shift + 8) & 255], ones)

                def body(c, carry):
                    nxt = []
                    for j in (0, 1):
                        emit(j, carry[j], oos[j])
                        nxt.append(fetch(j, c + 1))
                    return tuple(nxt)

                carry = tuple(fetch(j, 0) for j in (0, 1))
                carry = lax.fori_loop(0, _NCH - 1, body, carry)
                for j in (0, 1):
                    emit(j, carry[j], oos[j])

            for pno in range(4):
                lax.fori_loop(0, 16, prefix, (jnp.int32(0), jnp.int32(0)))
                lax.fori_loop(0, 16, zero_cnt, 0)
                run_scat(pno)
            return rcarry

        lax.fori_loop(0, _BATCH // 2, pair_body, 0)
        pltpu.sync_copy(outk_buf, outk_hbm.at[pl.ds(r0 * _W, _BATCH * _W)])
        pltpu.sync_copy(outp_buf, outp_hbm.at[pl.ds(r0 * _W, _BATCH * _W)])
        return carry0

    lax.fori_loop(0, _NB, batch_body, 0)


def kernel(z, x, y, x_mask, temporal_edge_index, temporal_edge_attr,
           sliding_wdw, W1, b1, W2, b2):
    n, dd = z.shape
    l = x.shape[1]
    merge_num = n // 2
    f32 = jnp.float32

    z_proj = pl.pallas_call(
        _mlp_kernel,
        grid=(n // _MLP_BLK,),
        in_specs=[
            pl.BlockSpec((_MLP_BLK, dd), lambda i: (i, 0)),
            pl.BlockSpec((dd, _HIDDEN), lambda i: (0, 0)),
            pl.BlockSpec((1, _HIDDEN), lambda i: (0, 0)),
            pl.BlockSpec((_HIDDEN, _HIDDEN), lambda i: (0, 0)),
            pl.BlockSpec((1, _HIDDEN), lambda i: (0, 0)),
        ],
        out_specs=pl.BlockSpec((_MLP_BLK, _HIDDEN), lambda i: (i, 0)),
        out_shape=jax.ShapeDtypeStruct((n, _HIDDEN), f32),
    )(z, W1, b1.reshape(1, -1), W2, b2.reshape(1, -1))

    zp_roll = jnp.roll(z_proj, -1, axis=0)
    z_sim = pl.pallas_call(
        functools.partial(_zsim_kernel, n, _MLP_BLK),
        grid=(n // _MLP_BLK,),
        in_specs=[
            pl.BlockSpec((_MLP_BLK, _HIDDEN), lambda i: (i, 0)),
            pl.BlockSpec((_MLP_BLK, _HIDDEN), lambda i: (i, 0)),
        ],
        out_specs=pl.BlockSpec((_MLP_BLK, 1), lambda i: (i, 0)),
        out_shape=jax.ShapeDtypeStruct((n, 1), f32),
    )(z_proj, zp_roll)

    comp = pl.pallas_call(
        functools.partial(_rank_kernel, n, _RANK_BLK, n - merge_num),
        grid=(n // _RANK_BLK,),
        in_specs=[
            pl.BlockSpec((_RANK_BLK, 1), lambda i: (i, 0)),
            pl.BlockSpec((1, n), lambda i: (0, 0)),
        ],
        out_specs=pl.BlockSpec((_RANK_BLK, 1), lambda i: (i, 0)),
        out_shape=jax.ShapeDtypeStruct((n, 1), f32),
    )(z_sim, z_sim.reshape(1, n))

    z_roll = jnp.roll(z, -1, axis=0)
    y2 = y.reshape(n, 1)
    y2_roll = jnp.roll(y2, -1, axis=0)
    z_n, y_merge2 = pl.pallas_call(
        _merge_kernel,
        grid=(n // _MERGE_BLK,),
        in_specs=[
            pl.BlockSpec((_MERGE_BLK, dd), lambda i: (i, 0)),
            pl.BlockSpec((_MERGE_BLK, dd), lambda i: (i, 0)),
            pl.BlockSpec((_MERGE_BLK, 1), lambda i: (i, 0)),
            pl.BlockSpec((_MERGE_BLK, 1), lambda i: (i, 0)),
            pl.BlockSpec((_MERGE_BLK, 1), lambda i: (i, 0)),
        ],
        out_specs=[
            pl.BlockSpec((_MERGE_BLK, dd), lambda i: (i, 0)),
            pl.BlockSpec((_MERGE_BLK, 1), lambda i: (i, 0)),
        ],
        out_shape=[
            jax.ShapeDtypeStruct((n, dd), f32),
            jax.ShapeDtypeStruct((n, 1), f32),
        ],
    )(z, z_roll, y2, y2_roll, comp)

    vals, idxs = pl.pallas_call(
        functools.partial(_knn_kernel, n, _KNN_BLK),
        grid=(n // _KNN_BLK,),
        in_specs=[
            pl.BlockSpec((_KNN_BLK, dd), lambda i: (i, 0)),
            pl.BlockSpec((n, dd), lambda i: (0, 0)),
        ],
        out_specs=[
            pl.BlockSpec((_KNN_BLK, _K + 1), lambda i: (i, 0)),
            pl.BlockSpec((_KNN_BLK, _K + 1), lambda i: (i, 0)),
        ],
        out_shape=[
            jax.ShapeDtypeStruct((n, _K + 1), f32),
            jax.ShapeDtypeStruct((n, _K + 1), jnp.int32),
        ],
    )(z_n, z_n)

    mm_flat, xm_flat = _sc_sort(x.reshape(-1), x_mask.reshape(-1), comp.reshape(-1))
    x_mask_merge = mm_flat.reshape(n, 2 * l)
    x_merge = xm_flat.reshape(n, 2 * l)

    dist = vals[:, 1:]
    idx = idxs[:, 1:]
    idx_source = jnp.repeat(jnp.arange(n, dtype=jnp.int32), _K)
    edge_index = jnp.stack([idx_source, idx.reshape(-1)], axis=0)
    attr = dist.reshape(-1, 1)
    return (x_merge, edge_index, attr, y_merge2.reshape(n),
            temporal_edge_index, temporal_edge_attr, x_mask_merge)
